# baseline (speedup 1.0000x reference)
import jax
import jax.numpy as jnp
from jax.experimental import pallas as pl
from jax.experimental.pallas import tpu as pltpu

_ALPHA = 0.2
_NEG = -9e15
_BLK = 256


def _gat_body(x_ref, adj_ref, w_ref, b_ref, a1_ref, a2_ref, out_ref,
              nf_ref, s2_ref):
    i = pl.program_id(0)

    @pl.when(i == 0)
    def _():
        nf = jax.lax.dot_general(
            x_ref[...], w_ref[...], (((1,), (1,)), ((), ())),
            preferred_element_type=jnp.float32,
        ) + b_ref[...]
        nf_ref[...] = nf
        s2_ref[...] = jax.lax.dot_general(
            a2_ref[...], nf, (((1,), (1,)), ((), ())),
            preferred_element_type=jnp.float32,
        )

    nf = nf_ref[...]
    nfb = nf_ref[pl.ds(i * _BLK, _BLK), :]
    s1 = jax.lax.dot_general(
        nfb, a1_ref[...], (((1,), (1,)), ((), ())),
        preferred_element_type=jnp.float32,
    )
    logits = s1 + s2_ref[...]
    leaky = jnp.where(logits >= 0, logits, _ALPHA * logits)
    masked = jnp.where(adj_ref[...] != 0, leaky, _NEG)
    m = jnp.max(masked, axis=1, keepdims=True)
    e = jnp.exp(masked - m)
    denom = jnp.sum(e, axis=1, keepdims=True)
    probs = e / denom
    out_ref[...] = jax.lax.dot_general(
        probs, nf, (((1,), (0,)), ((), ())),
        preferred_element_type=jnp.float32,
    )


def kernel(node_feats, adj_matrix, W, b, a):
    if node_feats.ndim == 2:
        node_feats = node_feats[None]
    B, N, C_IN = node_feats.shape
    C_OUT = W.shape[0]
    x = node_feats.reshape(N, C_IN)
    adj = adj_matrix.reshape(N, N)
    a1 = a[:, :C_OUT]
    a2 = a[:, C_OUT:]
    b2 = b.reshape(1, C_OUT)
    nblk = N // _BLK
    out = pl.pallas_call(
        _gat_body,
        grid=(nblk,),
        in_specs=[
            pl.BlockSpec((N, C_IN), lambda i: (0, 0)),
            pl.BlockSpec((_BLK, N), lambda i: (i, 0)),
            pl.BlockSpec((C_OUT, C_IN), lambda i: (0, 0)),
            pl.BlockSpec((1, C_OUT), lambda i: (0, 0)),
            pl.BlockSpec((1, C_OUT), lambda i: (0, 0)),
            pl.BlockSpec((1, C_OUT), lambda i: (0, 0)),
        ],
        out_specs=pl.BlockSpec((_BLK, C_OUT), lambda i: (i, 0)),
        out_shape=jax.ShapeDtypeStruct((N, C_OUT), jnp.float32),
        scratch_shapes=[
            pltpu.VMEM((N, C_OUT), jnp.float32),
            pltpu.VMEM((1, N), jnp.float32),
        ],
    )(x, adj, W, b2, a1, a2)
    return out.reshape(B, N, C_OUT)


# speedup vs baseline: 1.0952x; 1.0952x over previous
import jax
import jax.numpy as jnp
from jax.experimental import pallas as pl
from jax.experimental.pallas import tpu as pltpu

_ALPHA = 0.2
_NEG = -9e15
_BLK = 512


def _gat_body(x_ref, adj_ref, w_ref, b_ref, a1_ref, a2_ref, out_ref,
              nf_ref, s2_ref):
    i = pl.program_id(0)

    @pl.when(i == 0)
    def _():
        nf = jax.lax.dot_general(
            x_ref[...], w_ref[...], (((1,), (1,)), ((), ())),
            preferred_element_type=jnp.float32,
        ) + b_ref[...]
        nf_ref[...] = nf
        s2_ref[...] = jax.lax.dot_general(
            a2_ref[...], nf, (((1,), (1,)), ((), ())),
            preferred_element_type=jnp.float32,
        )

    nf = nf_ref[...]
    nfb = nf_ref[pl.ds(i * _BLK, _BLK), :]
    s1 = jax.lax.dot_general(
        nfb, a1_ref[...], (((1,), (1,)), ((), ())),
        preferred_element_type=jnp.float32,
    )
    logits = s1 + s2_ref[...]
    leaky = jnp.where(logits >= 0, logits, _ALPHA * logits)
    masked = jnp.where(adj_ref[...] != 0, leaky, _NEG)
    m = jnp.max(masked, axis=1, keepdims=True)
    e = jnp.exp(masked - m)
    denom = jnp.sum(e, axis=1, keepdims=True)
    probs = e / denom
    out_ref[...] = jax.lax.dot_general(
        probs, nf, (((1,), (0,)), ((), ())),
        preferred_element_type=jnp.float32,
    )


def kernel(node_feats, adj_matrix, W, b, a):
    if node_feats.ndim == 2:
        node_feats = node_feats[None]
    B, N, C_IN = node_feats.shape
    C_OUT = W.shape[0]
    x = node_feats.reshape(N, C_IN)
    adj = adj_matrix.reshape(N, N)
    a1 = a[:, :C_OUT]
    a2 = a[:, C_OUT:]
    b2 = b.reshape(1, C_OUT)
    nblk = N // _BLK
    out = pl.pallas_call(
        _gat_body,
        grid=(nblk,),
        in_specs=[
            pl.BlockSpec((N, C_IN), lambda i: (0, 0)),
            pl.BlockSpec((_BLK, N), lambda i: (i, 0)),
            pl.BlockSpec((C_OUT, C_IN), lambda i: (0, 0)),
            pl.BlockSpec((1, C_OUT), lambda i: (0, 0)),
            pl.BlockSpec((1, C_OUT), lambda i: (0, 0)),
            pl.BlockSpec((1, C_OUT), lambda i: (0, 0)),
        ],
        out_specs=pl.BlockSpec((_BLK, C_OUT), lambda i: (i, 0)),
        out_shape=jax.ShapeDtypeStruct((N, C_OUT), jnp.float32),
        scratch_shapes=[
            pltpu.VMEM((N, C_OUT), jnp.float32),
            pltpu.VMEM((1, N), jnp.float32),
        ],
    )(x, adj, W, b2, a1, a2)
    return out.reshape(B, N, C_OUT)


# speedup vs baseline: 1.1535x; 1.0533x over previous
import jax
import jax.numpy as jnp
from jax.experimental import pallas as pl
from jax.experimental.pallas import tpu as pltpu

_ALPHA = 0.2
_NEG = -9e15
_BLK = 256


def _gat_body(x_ref, adj_ref, w_ref, b_ref, a_ref, out_ref, nf_ref, s2_ref):
    i = pl.program_id(0)
    c_out = w_ref.shape[0]
    a1 = a_ref[:, :c_out]
    a2 = a_ref[:, c_out:]

    @pl.when(i == 0)
    def _():
        nf = jax.lax.dot_general(
            x_ref[0], w_ref[...], (((1,), (1,)), ((), ())),
            preferred_element_type=jnp.float32,
        ) + b_ref[...]
        nf_ref[...] = nf
        s2_ref[...] = jax.lax.dot_general(
            a2, nf, (((1,), (1,)), ((), ())),
            preferred_element_type=jnp.float32,
        )

    nf = nf_ref[...]
    nfb = nf_ref[pl.ds(i * _BLK, _BLK), :]
    s1 = jax.lax.dot_general(
        nfb, a1, (((1,), (1,)), ((), ())),
        preferred_element_type=jnp.float32,
    )
    logits = s1 + s2_ref[...]
    leaky = jnp.where(logits >= 0, logits, _ALPHA * logits)
    masked = jnp.where(adj_ref[0] != 0, leaky, _NEG)
    m = jnp.max(masked, axis=1, keepdims=True)
    e = jnp.exp(masked - m)
    denom = jnp.sum(e, axis=1, keepdims=True)
    probs = e / denom
    out_ref[0] = jax.lax.dot_general(
        probs, nf, (((1,), (0,)), ((), ())),
        preferred_element_type=jnp.float32,
    )


def kernel(node_feats, adj_matrix, W, b, a):
    if node_feats.ndim == 2:
        node_feats = node_feats[None]
    B, N, C_IN = node_feats.shape
    C_OUT = W.shape[0]
    nblk = N // _BLK
    out = pl.pallas_call(
        _gat_body,
        grid=(nblk,),
        in_specs=[
            pl.BlockSpec((1, N, C_IN), lambda i: (0, 0, 0)),
            pl.BlockSpec((1, _BLK, N), lambda i: (0, i, 0)),
            pl.BlockSpec((C_OUT, C_IN), lambda i: (0, 0)),
            pl.BlockSpec((C_OUT,), lambda i: (0,)),
            pl.BlockSpec((1, 2 * C_OUT), lambda i: (0, 0)),
        ],
        out_specs=pl.BlockSpec((1, _BLK, C_OUT), lambda i: (0, i, 0)),
        out_shape=jax.ShapeDtypeStruct((B, N, C_OUT), jnp.float32),
        scratch_shapes=[
            pltpu.VMEM((N, C_OUT), jnp.float32),
            pltpu.VMEM((1, N), jnp.float32),
        ],
    )(node_feats, adj_matrix, W, b, a)
    return out


# speedup vs baseline: 1.2412x; 1.0760x over previous
import jax
import jax.numpy as jnp
from jax.experimental import pallas as pl
from jax.experimental.pallas import tpu as pltpu

_ALPHA = 0.2
_NEG = -9e15
_BLK = 1024


def _gat_body(x_ref, adj_ref, w_ref, b_ref, a_ref, out_ref, nf_ref, s2_ref):
    i = pl.program_id(0)
    c_out = w_ref.shape[0]
    a1 = a_ref[:, :c_out]
    a2 = a_ref[:, c_out:]

    @pl.when(i == 0)
    def _():
        nf = jax.lax.dot_general(
            x_ref[0], w_ref[...], (((1,), (1,)), ((), ())),
            preferred_element_type=jnp.float32,
        ) + b_ref[...]
        nf_ref[...] = nf
        s2_ref[...] = jax.lax.dot_general(
            a2, nf, (((1,), (1,)), ((), ())),
            preferred_element_type=jnp.float32,
        )

    nf = nf_ref[...]
    nfb = nf_ref[pl.ds(i * _BLK, _BLK), :]
    s1 = jax.lax.dot_general(
        nfb, a1, (((1,), (1,)), ((), ())),
        preferred_element_type=jnp.float32,
    )
    logits = s1 + s2_ref[...]
    leaky = jnp.where(logits >= 0, logits, _ALPHA * logits)
    masked = jnp.where(adj_ref[0] != 0, leaky, _NEG)
    m = jnp.max(masked, axis=1, keepdims=True)
    e = jnp.exp(masked - m)
    denom = jnp.sum(e, axis=1, keepdims=True)
    probs = e / denom
    out_ref[0] = jax.lax.dot_general(
        probs, nf, (((1,), (0,)), ((), ())),
        preferred_element_type=jnp.float32,
    )


def kernel(node_feats, adj_matrix, W, b, a):
    if node_feats.ndim == 2:
        node_feats = node_feats[None]
    B, N, C_IN = node_feats.shape
    C_OUT = W.shape[0]
    nblk = N // _BLK
    out = pl.pallas_call(
        _gat_body,
        grid=(nblk,),
        in_specs=[
            pl.BlockSpec((1, N, C_IN), lambda i: (0, 0, 0)),
            pl.BlockSpec((1, _BLK, N), lambda i: (0, i, 0)),
            pl.BlockSpec((C_OUT, C_IN), lambda i: (0, 0)),
            pl.BlockSpec((C_OUT,), lambda i: (0,)),
            pl.BlockSpec((1, 2 * C_OUT), lambda i: (0, 0)),
        ],
        out_specs=pl.BlockSpec((1, _BLK, C_OUT), lambda i: (0, i, 0)),
        out_shape=jax.ShapeDtypeStruct((B, N, C_OUT), jnp.float32),
        scratch_shapes=[
            pltpu.VMEM((N, C_OUT), jnp.float32),
            pltpu.VMEM((1, N), jnp.float32),
        ],
    )(node_feats, adj_matrix, W, b, a)
    return out


# speedup vs baseline: 1.3416x; 1.0809x over previous
import jax
import jax.numpy as jnp
from jax.experimental import pallas as pl
from jax.experimental.pallas import tpu as pltpu

_ALPHA = 0.2
_NEG = -9e15
_BLK = 512


def _gat_body(x_ref, adj_ref, w_ref, b_ref, a_ref, out_ref, nf_ref, s2_ref):
    i = pl.program_id(0)
    c_out = w_ref.shape[0]
    a1 = a_ref[:, :c_out]
    a2 = a_ref[:, c_out:]

    @pl.when(i == 0)
    def _():
        nf = jax.lax.dot_general(
            x_ref[0], w_ref[...], (((1,), (1,)), ((), ())),
            preferred_element_type=jnp.float32,
        ) + b_ref[...]
        nf_ref[...] = nf
        s2_ref[...] = jax.lax.dot_general(
            a2, nf, (((1,), (1,)), ((), ())),
            preferred_element_type=jnp.float32,
        )

    nf = nf_ref[...]
    nfb = nf_ref[pl.ds(i * _BLK, _BLK), :]
    s1 = jax.lax.dot_general(
        nfb, a1, (((1,), (1,)), ((), ())),
        preferred_element_type=jnp.float32,
    )
    logits = s1 + s2_ref[...]
    leaky = jnp.maximum(logits, _ALPHA * logits)
    masked = jnp.where(adj_ref[0] != 0, leaky, _NEG)
    m = jnp.max(masked, axis=1, keepdims=True)
    e = jnp.exp(masked - m)
    denom = jnp.sum(e, axis=1, keepdims=True)
    acc = jax.lax.dot_general(
        e, nf, (((1,), (0,)), ((), ())),
        preferred_element_type=jnp.float32,
    )
    out_ref[0] = acc / denom


def kernel(node_feats, adj_matrix, W, b, a):
    if node_feats.ndim == 2:
        node_feats = node_feats[None]
    B, N, C_IN = node_feats.shape
    C_OUT = W.shape[0]
    nblk = N // _BLK
    out = pl.pallas_call(
        _gat_body,
        grid=(nblk,),
        in_specs=[
            pl.BlockSpec((1, N, C_IN), lambda i: (0, 0, 0)),
            pl.BlockSpec((1, _BLK, N), lambda i: (0, i, 0)),
            pl.BlockSpec((C_OUT, C_IN), lambda i: (0, 0)),
            pl.BlockSpec((C_OUT,), lambda i: (0,)),
            pl.BlockSpec((1, 2 * C_OUT), lambda i: (0, 0)),
        ],
        out_specs=pl.BlockSpec((1, _BLK, C_OUT), lambda i: (0, i, 0)),
        out_shape=jax.ShapeDtypeStruct((B, N, C_OUT), jnp.float32),
        scratch_shapes=[
            pltpu.VMEM((N, C_OUT), jnp.float32),
            pltpu.VMEM((1, N), jnp.float32),
        ],
    )(node_feats, adj_matrix, W, b, a)
    return out
